# Initial kernel scaffold; baseline (speedup 1.0000x reference)
import jax
import jax.numpy as jnp
from jax.experimental import pallas as pl


def kernel(inputs, hx, adj, W_gate, b_gate, W_c, b_c):
    raise NotImplementedError("write your pallas kernel here")


# speedup vs baseline: 5.7584x; 5.7584x over previous
import functools

import jax
import jax.numpy as jnp
from jax.experimental import pallas as pl
from jax.experimental.pallas import tpu as pltpu

N = 1024
F = 64
B = 16
BB = 4
M = 5


def _dotT(a, b):
    return jax.lax.dot_general(
        a, b, (((0,), (0,)), ((), ())), preferred_element_type=jnp.float32)


def _dot(a, b):
    return jax.lax.dot_general(
        a, b, (((1,), (0,)), ((), ())), preferred_element_type=jnp.float32)


def _cell_kernel(inp_ref, hx_ref, adj_ref, wg_ref, bg_ref, wc_ref, bc_ref,
                 out_ref, s1_ref, s2_ref):
    step = pl.program_id(0)

    @pl.when(step == 0)
    def _build_supports():
        a = adj_ref[...]
        d = jnp.sum(a, axis=1, keepdims=True)
        dinv = jnp.where(d > 0.0, 1.0 / d, 0.0)
        s1_ref[...] = dinv * a
        d2 = jnp.sum(a, axis=0, keepdims=True)
        d2inv = jnp.where(d2 > 0.0, 1.0 / d2, 0.0)
        s2_ref[...] = a * d2inv

    rw = s1_ref[...]
    s2 = s2_ref[...]

    inp2 = jnp.concatenate([inp_ref[b] for b in range(BB)], axis=1)
    st2 = jnp.concatenate([hx_ref[b] for b in range(BB)], axis=1)

    def diffuse(x):
        y1 = _dotT(rw, x)
        y2 = 2.0 * _dotT(rw, y1) - x
        z1 = _dot(s2, x)
        z2 = 2.0 * _dot(s2, z1) - x
        return y1, y2, z1, z2

    p1, p2, p3, p4 = diffuse(inp2)
    q1, q2, q3, q4 = diffuse(st2)

    wg = wg_ref[...]
    bg = bg_ref[...]
    wc = wc_ref[...]
    bc = bc_ref[...]

    st2p_parts = []
    u_parts = []
    for b in range(BB):
        lo, hi = b * F, (b + 1) * F
        xb = jnp.concatenate(
            [inp2[:, lo:hi], st2[:, lo:hi], p1[:, lo:hi], q1[:, lo:hi],
             p2[:, lo:hi], q2[:, lo:hi], p3[:, lo:hi], q3[:, lo:hi],
             p4[:, lo:hi], q4[:, lo:hi]], axis=1)
        val = jax.nn.sigmoid(_dot(xb, wg) + bg)
        u_parts.append(val[:, F:])
        st2p_parts.append(val[:, :F] * st2[:, lo:hi])

    st2p = jnp.concatenate(st2p_parts, axis=1)
    r1, r2, r3, r4 = diffuse(st2p)

    for b in range(BB):
        lo, hi = b * F, (b + 1) * F
        xb = jnp.concatenate(
            [inp2[:, lo:hi], st2p[:, lo:hi], p1[:, lo:hi], r1[:, lo:hi],
             p2[:, lo:hi], r2[:, lo:hi], p3[:, lo:hi], r3[:, lo:hi],
             p4[:, lo:hi], r4[:, lo:hi]], axis=1)
        c = jnp.tanh(_dot(xb, wc) + bc)
        u = u_parts[b]
        out_ref[b] = u * st2[:, lo:hi] + (1.0 - u) * c


def kernel(inputs, hx, adj, W_gate, b_gate, W_c, b_c):
    inp3 = inputs.reshape(B, N, F)
    hx3 = hx.reshape(B, N, F)
    wg = W_gate.reshape(2 * F, M, 2 * F).transpose(1, 0, 2).reshape(M * 2 * F, 2 * F)
    wc = W_c.reshape(2 * F, M, F).transpose(1, 0, 2).reshape(M * 2 * F, F)
    bg = b_gate.reshape(1, 2 * F)
    bc = b_c.reshape(1, F)

    out = pl.pallas_call(
        _cell_kernel,
        grid=(B // BB,),
        in_specs=[
            pl.BlockSpec((BB, N, F), lambda i: (i, 0, 0)),
            pl.BlockSpec((BB, N, F), lambda i: (i, 0, 0)),
            pl.BlockSpec((N, N), lambda i: (0, 0)),
            pl.BlockSpec((M * 2 * F, 2 * F), lambda i: (0, 0)),
            pl.BlockSpec((1, 2 * F), lambda i: (0, 0)),
            pl.BlockSpec((M * 2 * F, F), lambda i: (0, 0)),
            pl.BlockSpec((1, F), lambda i: (0, 0)),
        ],
        out_specs=pl.BlockSpec((BB, N, F), lambda i: (i, 0, 0)),
        out_shape=jax.ShapeDtypeStruct((B, N, F), jnp.float32),
        scratch_shapes=[
            pltpu.VMEM((N, N), jnp.float32),
            pltpu.VMEM((N, N), jnp.float32),
        ],
    )(inp3, hx3, adj, wg, bg, wc, bc)
    return out.reshape(B, N * F)


# speedup vs baseline: 5.8233x; 1.0113x over previous
import functools

import jax
import jax.numpy as jnp
from jax.experimental import pallas as pl
from jax.experimental.pallas import tpu as pltpu

N = 1024
F = 64
B = 16
BB = 4
M = 5


def _dotT(a, b):
    return jax.lax.dot_general(
        a, b, (((0,), (0,)), ((), ())), preferred_element_type=jnp.float32)


def _dot(a, b):
    return jax.lax.dot_general(
        a, b, (((1,), (0,)), ((), ())), preferred_element_type=jnp.float32)


def _cell_kernel(inp_ref, hx_ref, adj_ref, wg_ref, bg_ref, wc_ref, bc_ref,
                 out_ref, s1_ref, s2_ref):
    step = pl.program_id(0)

    @pl.when(step == 0)
    def _build_supports():
        a = adj_ref[...]
        d = jnp.sum(a, axis=1, keepdims=True)
        dinv = jnp.where(d > 0.0, 1.0 / d, 0.0)
        s1_ref[...] = (dinv * a).astype(jnp.bfloat16)
        d2 = jnp.sum(a, axis=0, keepdims=True)
        d2inv = jnp.where(d2 > 0.0, 1.0 / d2, 0.0)
        s2_ref[...] = (a * d2inv).astype(jnp.bfloat16)

    rw = s1_ref[...]
    s2 = s2_ref[...]
    bf = jnp.bfloat16

    inp2 = jnp.concatenate([inp_ref[b] for b in range(BB)], axis=1)
    st2 = jnp.concatenate([hx_ref[b] for b in range(BB)], axis=1)

    def diffuse(xb, x32):
        y1 = _dotT(rw, xb).astype(bf)
        y2 = (2.0 * _dotT(rw, y1) - x32).astype(bf)
        z1 = _dot(s2, xb).astype(bf)
        z2 = (2.0 * _dot(s2, z1) - x32).astype(bf)
        return y1, y2, z1, z2

    inp2b = inp2.astype(bf)
    st2b = st2.astype(bf)
    p1, p2, p3, p4 = diffuse(inp2b, inp2)
    q1, q2, q3, q4 = diffuse(st2b, st2)

    wg = wg_ref[...]
    bg = bg_ref[...]
    wc = wc_ref[...]
    bc = bc_ref[...]

    st2p_parts = []
    u_parts = []
    for b in range(BB):
        lo, hi = b * F, (b + 1) * F
        xb = jnp.concatenate(
            [inp2b[:, lo:hi], st2b[:, lo:hi], p1[:, lo:hi], q1[:, lo:hi],
             p2[:, lo:hi], q2[:, lo:hi], p3[:, lo:hi], q3[:, lo:hi],
             p4[:, lo:hi], q4[:, lo:hi]], axis=1)
        val = jax.nn.sigmoid(_dot(xb, wg) + bg)
        u_parts.append(val[:, F:])
        st2p_parts.append(val[:, :F] * st2[:, lo:hi])

    st2p = jnp.concatenate(st2p_parts, axis=1)
    st2pb = st2p.astype(bf)
    r1, r2, r3, r4 = diffuse(st2pb, st2p)

    for b in range(BB):
        lo, hi = b * F, (b + 1) * F
        xb = jnp.concatenate(
            [inp2b[:, lo:hi], st2pb[:, lo:hi], p1[:, lo:hi], r1[:, lo:hi],
             p2[:, lo:hi], r2[:, lo:hi], p3[:, lo:hi], r3[:, lo:hi],
             p4[:, lo:hi], r4[:, lo:hi]], axis=1)
        c = jnp.tanh(_dot(xb, wc) + bc)
        u = u_parts[b]
        out_ref[b] = u * st2[:, lo:hi] + (1.0 - u) * c


def kernel(inputs, hx, adj, W_gate, b_gate, W_c, b_c):
    inp3 = inputs.reshape(B, N, F)
    hx3 = hx.reshape(B, N, F)
    wg = W_gate.reshape(2 * F, M, 2 * F).transpose(1, 0, 2).reshape(
        M * 2 * F, 2 * F).astype(jnp.bfloat16)
    wc = W_c.reshape(2 * F, M, F).transpose(1, 0, 2).reshape(
        M * 2 * F, F).astype(jnp.bfloat16)
    bg = b_gate.reshape(1, 2 * F)
    bc = b_c.reshape(1, F)

    out = pl.pallas_call(
        _cell_kernel,
        grid=(B // BB,),
        in_specs=[
            pl.BlockSpec((BB, N, F), lambda i: (i, 0, 0)),
            pl.BlockSpec((BB, N, F), lambda i: (i, 0, 0)),
            pl.BlockSpec((N, N), lambda i: (0, 0)),
            pl.BlockSpec((M * 2 * F, 2 * F), lambda i: (0, 0)),
            pl.BlockSpec((1, 2 * F), lambda i: (0, 0)),
            pl.BlockSpec((M * 2 * F, F), lambda i: (0, 0)),
            pl.BlockSpec((1, F), lambda i: (0, 0)),
        ],
        out_specs=pl.BlockSpec((BB, N, F), lambda i: (i, 0, 0)),
        out_shape=jax.ShapeDtypeStruct((B, N, F), jnp.float32),
        scratch_shapes=[
            pltpu.VMEM((N, N), jnp.bfloat16),
            pltpu.VMEM((N, N), jnp.bfloat16),
        ],
    )(inp3, hx3, adj, wg, bg, wc, bc)
    return out.reshape(B, N * F)


# speedup vs baseline: 6.1054x; 1.0484x over previous
import jax
import jax.numpy as jnp
from jax.experimental import pallas as pl
from jax.experimental.pallas import tpu as pltpu

N = 1024
F = 64
B = 16
BB = 8
M = 5


def _dotT(a, b):
    return jax.lax.dot_general(
        a, b, (((0,), (0,)), ((), ())), preferred_element_type=jnp.float32)


def _dot(a, b):
    return jax.lax.dot_general(
        a, b, (((1,), (0,)), ((), ())), preferred_element_type=jnp.float32)


def _supports_kernel(adj_ref, s1_ref, s2_ref):
    a = adj_ref[...]
    d = jnp.sum(a, axis=1, keepdims=True)
    dinv = jnp.where(d > 0.0, 1.0 / d, 0.0)
    s1_ref[...] = (dinv * a).astype(jnp.bfloat16)
    d2 = jnp.sum(a, axis=0, keepdims=True)
    d2inv = jnp.where(d2 > 0.0, 1.0 / d2, 0.0)
    s2_ref[...] = (a * d2inv).astype(jnp.bfloat16)


def _cell_kernel(inp_ref, hx_ref, s1_ref, s2_ref, wg_ref, bg_ref, wc_ref,
                 bc_ref, out_ref):
    rw = s1_ref[...]
    s2 = s2_ref[...]
    bf = jnp.bfloat16

    inp2b = jnp.concatenate(
        [inp_ref[b].astype(bf) for b in range(BB)], axis=1)
    st2 = jnp.concatenate([hx_ref[b] for b in range(BB)], axis=1)
    st2b = st2.astype(bf)

    def diffuse(xb, x32):
        y1 = _dotT(rw, xb).astype(bf)
        y2 = (2.0 * _dotT(rw, y1) - x32).astype(bf)
        z1 = _dot(s2, xb).astype(bf)
        z2 = (2.0 * _dot(s2, z1) - x32).astype(bf)
        return y1, y2, z1, z2

    p1, p2, p3, p4 = diffuse(inp2b, inp2b.astype(jnp.float32))
    q1, q2, q3, q4 = diffuse(st2b, st2)

    wg = wg_ref[...]
    bg = bg_ref[...]
    wc = wc_ref[...]
    bc = bc_ref[...]

    st2p_parts = []
    u_parts = []
    for b in range(BB):
        lo, hi = b * F, (b + 1) * F
        xb = jnp.concatenate(
            [inp2b[:, lo:hi], st2b[:, lo:hi], p1[:, lo:hi], q1[:, lo:hi],
             p2[:, lo:hi], q2[:, lo:hi], p3[:, lo:hi], q3[:, lo:hi],
             p4[:, lo:hi], q4[:, lo:hi]], axis=1)
        val = jax.nn.sigmoid(_dot(xb, wg) + bg)
        u_parts.append(val[:, F:])
        st2p_parts.append(val[:, :F] * st2[:, lo:hi])

    st2p = jnp.concatenate(st2p_parts, axis=1)
    st2pb = st2p.astype(bf)
    r1, r2, r3, r4 = diffuse(st2pb, st2p)

    for b in range(BB):
        lo, hi = b * F, (b + 1) * F
        xb = jnp.concatenate(
            [inp2b[:, lo:hi], st2pb[:, lo:hi], p1[:, lo:hi], r1[:, lo:hi],
             p2[:, lo:hi], r2[:, lo:hi], p3[:, lo:hi], r3[:, lo:hi],
             p4[:, lo:hi], r4[:, lo:hi]], axis=1)
        c = jnp.tanh(_dot(xb, wc) + bc)
        u = u_parts[b]
        out_ref[b] = u * st2[:, lo:hi] + (1.0 - u) * c


def kernel(inputs, hx, adj, W_gate, b_gate, W_c, b_c):
    inp3 = inputs.reshape(B, N, F)
    hx3 = hx.reshape(B, N, F)
    wg = W_gate.reshape(2 * F, M, 2 * F).transpose(1, 0, 2).reshape(
        M * 2 * F, 2 * F).astype(jnp.bfloat16)
    wc = W_c.reshape(2 * F, M, F).transpose(1, 0, 2).reshape(
        M * 2 * F, F).astype(jnp.bfloat16)
    bg = b_gate.reshape(1, 2 * F)
    bc = b_c.reshape(1, F)

    s1, s2 = pl.pallas_call(
        _supports_kernel,
        out_shape=[
            jax.ShapeDtypeStruct((N, N), jnp.bfloat16),
            jax.ShapeDtypeStruct((N, N), jnp.bfloat16),
        ],
    )(adj)

    out = pl.pallas_call(
        _cell_kernel,
        grid=(B // BB,),
        in_specs=[
            pl.BlockSpec((BB, N, F), lambda i: (i, 0, 0)),
            pl.BlockSpec((BB, N, F), lambda i: (i, 0, 0)),
            pl.BlockSpec((N, N), lambda i: (0, 0)),
            pl.BlockSpec((N, N), lambda i: (0, 0)),
            pl.BlockSpec((M * 2 * F, 2 * F), lambda i: (0, 0)),
            pl.BlockSpec((1, 2 * F), lambda i: (0, 0)),
            pl.BlockSpec((M * 2 * F, F), lambda i: (0, 0)),
            pl.BlockSpec((1, F), lambda i: (0, 0)),
        ],
        out_specs=pl.BlockSpec((BB, N, F), lambda i: (i, 0, 0)),
        out_shape=jax.ShapeDtypeStruct((B, N, F), jnp.float32),
    )(inp3, hx3, s1, s2, wg, bg, wc, bc)
    return out.reshape(B, N * F)
